# Initial kernel scaffold; baseline (speedup 1.0000x reference)
import jax
import jax.numpy as jnp
from jax.experimental import pallas as pl


def kernel(x, table):
    raise NotImplementedError("write your pallas kernel here")


# speedup vs baseline: 8.6688x; 8.6688x over previous
import functools

import jax
import jax.numpy as jnp
from jax import lax
from jax.experimental import pallas as pl
from jax.experimental.pallas import tpu as pltpu
from jax.experimental.pallas import tpu_sc as plsc

D_EMBED = 128
TABLE = 32
N = 262144

NC = 2
NS = 16
NW = NC * NS
PTS = N // NW
B = 64
NBATCH = PTS // B
LANES = 16
CH = D_EMBED // LANES


def _axis_terms(v, stride):
    iv = v * (TABLE / 2.0) + (TABLE - 1.0) / 2.0
    i0 = (iv + 1.0).astype(jnp.int32) - 1
    f0 = i0.astype(jnp.float32)
    w1 = iv - f0
    w0 = 1.0 - w1
    w0 = jnp.where(i0 >= 0, w0, 0.0)
    w1 = jnp.where(i0 <= TABLE - 2, w1, 0.0)
    e0 = jnp.maximum(i0, 0) * stride
    e1 = jnp.minimum(i0 + 1, TABLE - 1) * stride
    return e0, e1, w0, w1


def _body(xs_hbm, ys_hbm, zs_hbm, tab_hbm, out_hbm,
          xs_v, ys_v, zs_v, idx_v, w_v, rows_v, out_v, sem):
    c = lax.axis_index("c")
    s = lax.axis_index("s")
    wid = s * NC + c
    base = wid * PTS

    pltpu.sync_copy(xs_hbm.at[pl.ds(base, PTS)], xs_v)
    pltpu.sync_copy(ys_hbm.at[pl.ds(base, PTS)], ys_v)
    pltpu.sync_copy(zs_hbm.at[pl.ds(base, PTS)], zs_v)

    def batch_body(g, carry):
        b0 = g * B
        for i in range(B // LANES):
            src = pl.ds(b0 + i * LANES, LANES)
            dst = pl.ds(i * LANES, LANES)
            xv = jnp.clip(xs_v[src], -1.0, 1.0)
            yv = jnp.clip(ys_v[src], -1.0, 1.0)
            zv = jnp.clip(zs_v[src], -1.0, 1.0)
            ex0, ex1, wx0, wx1 = _axis_terms(xv, 1)
            ey0, ey1, wy0, wy1 = _axis_terms(yv, TABLE)
            ez0, ez1, wz0, wz1 = _axis_terms(zv, TABLE * TABLE)
            for k in range(8):
                ex, wx = (ex1, wx1) if (k & 1) else (ex0, wx0)
                ey, wy = (ey1, wy1) if (k & 2) else (ey0, wy0)
                ez, wz = (ez1, wz1) if (k & 4) else (ez0, wz0)
                idx_v[k, dst] = ez + ey + ex
                w_v[k, dst] = wz * wy * wx
        copies = [
            pltpu.async_copy(tab_hbm.at[idx_v.at[k]], rows_v.at[k], sem)
            for k in range(8)
        ]
        for cp in copies:
            cp.wait()

        def pt_body(b, _):
            lane = jnp.full((LANES,), lax.rem(b, LANES), jnp.int32)
            blk = lax.mul(lax.div(b, LANES), LANES)
            accs = [jnp.zeros((LANES,), jnp.float32)] * CH
            for k in range(8):
                wrow = w_v[k, pl.ds(blk, LANES)]
                wk = wrow.at[lane].get(mode="promise_in_bounds")
                for t in range(CH):
                    row = rows_v[k, b, pl.ds(t * LANES, LANES)]
                    accs[t] = accs[t] + wk * row
            for t in range(CH):
                out_v[b, pl.ds(t * LANES, LANES)] = accs[t]
            return 0

        lax.fori_loop(0, B, pt_body, 0)
        pltpu.sync_copy(out_v, out_hbm.at[pl.ds(base + b0, B)])
        return carry

    lax.fori_loop(0, NBATCH, batch_body, 0)


@functools.partial(jax.jit, donate_argnums=())
def kernel(x, table):
    xc = jnp.clip(x, -1.0, 1.0)
    xs = xc[:, 0]
    ys = xc[:, 1]
    zs = xc[:, 2]
    tab = table.reshape(D_EMBED, -1).T

    run = functools.partial(
        pl.kernel,
        out_type=jax.ShapeDtypeStruct((N, D_EMBED), jnp.float32),
        mesh=plsc.VectorSubcoreMesh(core_axis_name="c", subcore_axis_name="s"),
        scratch_types=[
            pltpu.VMEM((PTS,), jnp.float32),
            pltpu.VMEM((PTS,), jnp.float32),
            pltpu.VMEM((PTS,), jnp.float32),
            pltpu.VMEM((8, B), jnp.int32),
            pltpu.VMEM((8, B), jnp.float32),
            pltpu.VMEM((8, B, D_EMBED), jnp.float32),
            pltpu.VMEM((B, D_EMBED), jnp.float32),
            pltpu.SemaphoreType.DMA,
        ],
    )(_body)
    return run(xs, ys, zs, tab)


# speedup vs baseline: 15.0908x; 1.7408x over previous
import functools

import jax
import jax.numpy as jnp
from jax import lax
from jax.experimental import pallas as pl
from jax.experimental.pallas import tpu as pltpu
from jax.experimental.pallas import tpu_sc as plsc

D_EMBED = 128
TABLE = 32
N = 262144

NC = 2
NS = 16
NW = NC * NS
PTS = N // NW
B = 32
NBATCH = PTS // B
LANES = 16
CH = D_EMBED // LANES


def _axis_terms(v, stride):
    iv = v * (TABLE / 2.0) + (TABLE - 1.0) / 2.0
    i0 = (iv + 1.0).astype(jnp.int32) - 1
    f0 = i0.astype(jnp.float32)
    w1 = iv - f0
    w0 = 1.0 - w1
    w0 = jnp.where(i0 >= 0, w0, 0.0)
    w1 = jnp.where(i0 <= TABLE - 2, w1, 0.0)
    e0 = jnp.maximum(i0, 0) * stride
    e1 = jnp.minimum(i0 + 1, TABLE - 1) * stride
    return e0, e1, w0, w1


def _body(xs_hbm, ys_hbm, zs_hbm, tab_hbm, out_hbm,
          xs_v, ys_v, zs_v, idx_v, w_v, rows_v, out_v,
          gsem0, gsem1, osem0, osem1):
    c = lax.axis_index("c")
    s = lax.axis_index("s")
    wid = s * NC + c
    base = wid * PTS
    gsem = (gsem0, gsem1)
    osem = (osem0, osem1)

    pltpu.sync_copy(xs_hbm.at[pl.ds(base, PTS)], xs_v)
    pltpu.sync_copy(ys_hbm.at[pl.ds(base, PTS)], ys_v)
    pltpu.sync_copy(zs_hbm.at[pl.ds(base, PTS)], zs_v)

    def compute_and_fire(i, p):
        b0 = i * B
        for sl in range(B // LANES):
            src = pl.ds(b0 + sl * LANES, LANES)
            dst = pl.ds(sl * LANES, LANES)
            xv = jnp.clip(xs_v[src], -1.0, 1.0)
            yv = jnp.clip(ys_v[src], -1.0, 1.0)
            zv = jnp.clip(zs_v[src], -1.0, 1.0)
            ex0, ex1, wx0, wx1 = _axis_terms(xv, 1)
            ey0, ey1, wy0, wy1 = _axis_terms(yv, TABLE)
            ez0, ez1, wz0, wz1 = _axis_terms(zv, TABLE * TABLE)
            for k in range(8):
                ex, wx = (ex1, wx1) if (k & 1) else (ex0, wx0)
                ey, wy = (ey1, wy1) if (k & 2) else (ey0, wy0)
                ez, wz = (ez1, wz1) if (k & 4) else (ez0, wz0)
                idx_v[p, k, dst] = ez + ey + ex
                w_v[p, k, dst] = wz * wy * wx
        for k in range(8):
            pltpu.async_copy(tab_hbm.at[idx_v.at[p, k]], rows_v.at[p, k],
                             gsem[p])

    def wait_gathers(p):
        for k in range(8):
            pltpu.make_async_copy(tab_hbm.at[idx_v.at[p, k]],
                                  rows_v.at[p, k], gsem[p]).wait()

    def wait_out(i, p):
        pltpu.make_async_copy(out_v.at[p], out_hbm.at[pl.ds(base + i * B, B)],
                              osem[p]).wait()

    def blend(i, p):
        def pt_body(b, _):
            lane = jnp.full((LANES,), lax.rem(b, LANES), jnp.int32)
            blk = lax.mul(lax.div(b, LANES), LANES)
            accs = [jnp.zeros((LANES,), jnp.float32)] * CH
            for k in range(8):
                wrow = w_v[p, k, pl.ds(blk, LANES)]
                wk = wrow.at[lane].get(mode="promise_in_bounds")
                for t in range(CH):
                    row = rows_v[p, k, b, pl.ds(t * LANES, LANES)]
                    accs[t] = accs[t] + wk * row
            for t in range(CH):
                out_v[p, b, pl.ds(t * LANES, LANES)] = accs[t]
            return 0

        lax.fori_loop(0, B, pt_body, 0)

    def half(i, p):
        @pl.when(i + 1 < NBATCH)
        def _():
            compute_and_fire(i + 1, 1 - p)

        wait_gathers(p)

        @pl.when(i >= 2)
        def _():
            wait_out(i, p)

        blend(i, p)
        pltpu.async_copy(out_v.at[p], out_hbm.at[pl.ds(base + i * B, B)],
                         osem[p])

    def round_body(g, carry):
        half(2 * g, 0)
        half(2 * g + 1, 1)
        return carry

    compute_and_fire(0, 0)
    lax.fori_loop(0, NBATCH // 2, round_body, 0)
    wait_out(NBATCH - 2, 0)
    wait_out(NBATCH - 1, 1)


@functools.partial(jax.jit, donate_argnums=())
def kernel(x, table):
    xc = jnp.clip(x, -1.0, 1.0)
    xs = xc[:, 0]
    ys = xc[:, 1]
    zs = xc[:, 2]
    tab = table.reshape(D_EMBED, -1).T

    run = functools.partial(
        pl.kernel,
        out_type=jax.ShapeDtypeStruct((N, D_EMBED), jnp.float32),
        mesh=plsc.VectorSubcoreMesh(core_axis_name="c", subcore_axis_name="s"),
        scratch_types=[
            pltpu.VMEM((PTS,), jnp.float32),
            pltpu.VMEM((PTS,), jnp.float32),
            pltpu.VMEM((PTS,), jnp.float32),
            pltpu.VMEM((2, 8, B), jnp.int32),
            pltpu.VMEM((2, 8, B), jnp.float32),
            pltpu.VMEM((2, 8, B, D_EMBED), jnp.float32),
            pltpu.VMEM((2, B, D_EMBED), jnp.float32),
            pltpu.SemaphoreType.DMA,
            pltpu.SemaphoreType.DMA,
            pltpu.SemaphoreType.DMA,
            pltpu.SemaphoreType.DMA,
        ],
    )(_body)
    return run(xs, ys, zs, tab)
